# baseline (speedup 1.0000x reference)
import functools

import jax
import jax.numpy as jnp
from jax import lax
from jax.experimental import pallas as pl
from jax.experimental.pallas import tpu as pltpu
from jax.experimental.pallas import tpu_sc as plsc

_THR = 0.05
_TEMP = 2.0
_KL_WEIGHT = 0.2

_NS = 8192
_NT = 8192
_C = 22

_SBLK = 1024
_TCHUNK = 1024
_RBLK = 1024


def _nn_body(s_ref, tt_ref, oi_ref, od_ref):
    s3x2 = s_ref[...] * 2.0
    sx = s_ref[:, 0:1]
    sy = s_ref[:, 1:2]
    sz = s_ref[:, 2:3]
    s2 = sx * sx + sy * sy + sz * sz
    iota = lax.broadcasted_iota(jnp.int32, (_SBLK, _TCHUNK), 1).astype(
        jnp.float32)
    run_min = jnp.full((_SBLK, 1), jnp.inf, jnp.float32)
    run_idx = jnp.zeros((_SBLK, 1), jnp.float32)
    for c in range(_NT // _TCHUNK):
        cols = pl.ds(c * _TCHUNK, _TCHUNK)
        tx = tt_ref[0:1, cols]
        ty = tt_ref[1:2, cols]
        tz = tt_ref[2:3, cols]
        t2 = tx * tx + ty * ty + tz * tz
        dot2 = lax.dot_general(s3x2, tt_ref[:, cols],
                               (((1,), (0,)), ((), ())),
                               preferred_element_type=jnp.float32)
        d2 = s2 - dot2 + t2
        cmin = jnp.min(d2, axis=1, keepdims=True)
        cidx = jnp.min(jnp.where(d2 == cmin, iota, float(_NT)), axis=1,
                       keepdims=True) + float(_TCHUNK) * c
        better = cmin < run_min
        run_idx = jnp.where(better, cidx, run_idx)
        run_min = jnp.minimum(run_min, cmin)
    oi_ref[...] = run_idx.astype(jnp.int32)
    od_ref[...] = run_min


def _nn_cols(s_coord, t_coord_t, nrows):
    return pl.pallas_call(
        _nn_body,
        grid=(nrows // _SBLK,),
        in_specs=[
            pl.BlockSpec((_SBLK, 3), lambda i: (i, 0)),
            pl.BlockSpec((3, _NT), lambda i: (0, 0)),
        ],
        out_specs=[pl.BlockSpec((_SBLK, 1), lambda i: (i, 0)),
                   pl.BlockSpec((_SBLK, 1), lambda i: (i, 0))],
        out_shape=[jax.ShapeDtypeStruct((nrows, 1), jnp.int32),
                   jax.ShapeDtypeStruct((nrows, 1), jnp.float32)],
        compiler_params=pltpu.CompilerParams(
            dimension_semantics=("arbitrary",)),
    )(s_coord, t_coord_t)


def _make_sc_gather(nrows):
    info = plsc.get_sparse_core_info()
    nw = info.num_cores * info.num_subcores
    b_per_w = nrows // nw
    mesh = plsc.VectorSubcoreMesh(core_axis_name="c", subcore_axis_name="s")

    @functools.partial(
        pl.kernel,
        out_type=jax.ShapeDtypeStruct((nrows, 128), jnp.float32),
        mesh=mesh,
        scratch_types=[pltpu.VMEM((b_per_w,), jnp.int32),
                       pltpu.VMEM((b_per_w, 128), jnp.float32),
                       pltpu.SemaphoreType.DMA],
    )
    def gather_kernel(tl_hbm, idx_hbm, gl_hbm, idx_v, rows_l, sem_l):
        wid = lax.axis_index("s") * info.num_cores + lax.axis_index("c")
        base = wid * b_per_w
        pltpu.sync_copy(idx_hbm.at[pl.ds(base, b_per_w)], idx_v)
        pltpu.async_copy(tl_hbm.at[idx_v], rows_l, sem_l).wait()
        pltpu.sync_copy(rows_l, gl_hbm.at[pl.ds(base, b_per_w)])

    return gather_kernel


def _kl_body(d2a_ref, d2b_ref, sl_ref, gla_ref, glb_ref, o_ref):
    kl_sum = jnp.zeros((1, 1), jnp.float32)
    n_sum = jnp.zeros((1, 1), jnp.float32)
    inv_t = 1.0 / _TEMP
    half = _NS // 2
    for c in range(_NS // _RBLK):
        rows = pl.ds(c * _RBLK, _RBLK)
        if c < (_NS // _RBLK) // 2:
            hrows = pl.ds(c * _RBLK, _RBLK)
            d2_ref, gl_ref = d2a_ref, gla_ref
        else:
            hrows = pl.ds(c * _RBLK - half, _RBLK)
            d2_ref, gl_ref = d2b_ref, glb_ref
        dist = jnp.sqrt(jnp.maximum(d2_ref[hrows, :], 0.0))
        maskf = (dist <= _THR).astype(jnp.float32)
        sl = sl_ref[rows, :] * inv_t
        tl = gl_ref[hrows, 0:_C] * inv_t
        sm = jnp.max(sl, axis=1, keepdims=True)
        s_lse = jnp.log(jnp.sum(jnp.exp(sl - sm), axis=1, keepdims=True)) + sm
        tm = jnp.max(tl, axis=1, keepdims=True)
        te = jnp.exp(tl - tm)
        tsum = jnp.sum(te, axis=1, keepdims=True)
        t_lse = jnp.log(tsum) + tm
        t_prob = te / tsum
        kl_per = jnp.sum(t_prob * ((tl - t_lse) - (sl - s_lse)),
                         axis=1, keepdims=True)
        kl_sum = kl_sum + jnp.sum(kl_per * maskf, keepdims=True)
        n_sum = n_sum + jnp.sum(maskf, keepdims=True)
    loss = jnp.where(n_sum > 0.0, kl_sum / jnp.maximum(n_sum, 1.0), 0.0)
    o_ref[...] = loss * (_TEMP * _TEMP * _KL_WEIGHT)


def _kl_loss(d2a, d2b, s_logits, gl_a, gl_b):
    h = _NS // 2
    return pl.pallas_call(
        _kl_body,
        in_specs=[
            pl.BlockSpec((h, 1), lambda: (0, 0)),
            pl.BlockSpec((h, 1), lambda: (0, 0)),
            pl.BlockSpec((_NS, _C), lambda: (0, 0)),
            pl.BlockSpec((h, 128), lambda: (0, 0)),
            pl.BlockSpec((h, 128), lambda: (0, 0)),
        ],
        out_specs=pl.BlockSpec((1, 1), lambda: (0, 0)),
        out_shape=jax.ShapeDtypeStruct((1, 1), jnp.float32),
    )(d2a, d2b, s_logits, gl_a, gl_b)


def kernel(s_coord, t_coord, s_logits, t_logits):
    h = _NS // 2
    tt = t_coord.T
    tl_pad = jnp.pad(t_logits, ((0, 0), (0, 128 - _C)))
    gather = _make_sc_gather(h)
    col_a, d2a = _nn_cols(s_coord[:h], tt, h)
    gl_a = gather(tl_pad, col_a.reshape(h))
    col_b, d2b = _nn_cols(s_coord[h:], tt, h)
    gl_b = gather(tl_pad, col_b.reshape(h))
    out = _kl_loss(d2a, d2b, s_logits, gl_a, gl_b)
    return out[0, 0]
